# baseline (speedup 1.0000x reference)
import functools
import math

import jax
import jax.numpy as jnp
from jax.experimental import pallas as pl
from jax.experimental.pallas import tpu as pltpu
from jax.experimental.pallas import tpu_sc as plsc

_DIM = 64
_SCALE = math.sqrt(_DIM)
_LANES = 16
_W = 256
_NBUF = 5
_LEAD = 3


def kernel(x, lut):
    batch_shape = x.shape
    n = x.size
    info = plsc.get_sparse_core_info()
    nw = info.num_cores * info.num_subcores
    n_win = n // _W
    n_chunk = n_win // nw
    per_tile = n_chunk * _W

    idx = x.reshape(nw, per_tile).astype(jnp.int32)
    lut_s = lut * jnp.float32(_SCALE)

    mesh = plsc.VectorSubcoreMesh(
        core_axis_name="core", subcore_axis_name="subcore"
    )

    @functools.partial(
        pl.kernel,
        out_type=jax.ShapeDtypeStruct((n_win, _W, _DIM), jnp.float32),
        mesh=mesh,
        compiler_params=pltpu.CompilerParams(use_tc_tiling_on_sc=False),
        scratch_types=[
            pltpu.VMEM((per_tile,), jnp.int32),
            pltpu.VMEM((_NBUF, _W, _DIM), jnp.float32),
            pltpu.SemaphoreType.DMA((_NBUF,)),
            pltpu.SemaphoreType.DMA((_NBUF,)),
        ],
    )
    def emb(lut_hbm, i_hbm, o_hbm, idx_v, rows_v, sem_g, sem_w):
        wid = (
            jax.lax.axis_index("subcore") * info.num_cores
            + jax.lax.axis_index("core")
        )
        win0 = wid * n_chunk

        pltpu.sync_copy(i_hbm.at[wid], idx_v)

        def gather(c, b):
            for k in range(_W // _LANES):
                v = idx_v[pl.ds(c * _W + k * _LANES, _LANES)]
                pltpu.async_copy(
                    lut_hbm.at[v],
                    rows_v.at[b, pl.ds(k * _LANES, _LANES)],
                    sem_g.at[b],
                )

        def wait_gather(c, b):
            for k in range(_W // _LANES):
                v = idx_v[pl.ds(c * _W + k * _LANES, _LANES)]
                pltpu.make_async_copy(
                    lut_hbm.at[v],
                    rows_v.at[b, pl.ds(k * _LANES, _LANES)],
                    sem_g.at[b],
                ).wait()

        def write(c, b):
            pltpu.async_copy(
                rows_v.at[b], o_hbm.at[win0 + c], sem_w.at[b]
            )

        def wait_write(c, b):
            pltpu.make_async_copy(
                rows_v.at[b], o_hbm.at[win0 + c], sem_w.at[b]
            ).wait()

        for c in range(_LEAD):
            gather(c, c % _NBUF)

        @pl.loop(0, n_chunk, step=_NBUF)
        def _(jj):
            for bb in range(_NBUF):
                c = jj + bb
                b = bb
                bn = (b + _LEAD) % _NBUF

                @pl.when(c >= _NBUF - _LEAD)
                def _():
                    wait_write(c + _LEAD - _NBUF, bn)

                @pl.when(c + _LEAD < n_chunk)
                def _():
                    gather(c + _LEAD, bn)

                wait_gather(c, b)
                write(c, b)

        for c in range(n_chunk - (_NBUF - _LEAD), n_chunk):
            wait_write(c, c % _NBUF)

    out = emb(lut_s, idx)
    return out.reshape(*batch_shape, _DIM)


# speedup vs baseline: 1.2584x; 1.2584x over previous
import functools
import math

import jax
import jax.numpy as jnp
from jax.experimental import pallas as pl
from jax.experimental.pallas import tpu as pltpu
from jax.experimental.pallas import tpu_sc as plsc

_DIM = 64
_SCALE = math.sqrt(_DIM)
_LANES = 16
_W = 256
_NBUF = 5
_LEAD = 3


def kernel(x, lut):
    batch_shape = x.shape
    n = x.size
    info = plsc.get_sparse_core_info()
    nw = info.num_cores * info.num_subcores
    n_win = n // _W
    n_chunk = n_win // nw
    per_tile = n_chunk * _W

    idx = x.reshape(nw, per_tile).astype(jnp.int32)

    mesh = plsc.VectorSubcoreMesh(
        core_axis_name="core", subcore_axis_name="subcore"
    )

    @functools.partial(
        pl.kernel,
        out_type=jax.ShapeDtypeStruct((n_win, _W, _DIM), jnp.float32),
        mesh=mesh,
        compiler_params=pltpu.CompilerParams(use_tc_tiling_on_sc=False),
        scratch_types=[
            pltpu.VMEM((per_tile,), jnp.int32),
            pltpu.VMEM((_NBUF, _W, _DIM), jnp.float32),
            pltpu.SemaphoreType.DMA((_NBUF,)),
            pltpu.SemaphoreType.DMA((_NBUF,)),
        ],
    )
    def emb(lut_hbm, i_hbm, o_hbm, idx_v, rows_v, sem_g, sem_w):
        wid = (
            jax.lax.axis_index("subcore") * info.num_cores
            + jax.lax.axis_index("core")
        )
        win0 = wid * n_chunk

        pltpu.sync_copy(i_hbm.at[wid], idx_v)

        def gather(c, b):
            for k in range(_W // _LANES):
                v = idx_v[pl.ds(c * _W + k * _LANES, _LANES)]
                pltpu.async_copy(
                    lut_hbm.at[v],
                    rows_v.at[b, pl.ds(k * _LANES, _LANES)],
                    sem_g.at[b],
                )

        def wait_gather(c, b):
            for k in range(_W // _LANES):
                v = idx_v[pl.ds(c * _W + k * _LANES, _LANES)]
                pltpu.make_async_copy(
                    lut_hbm.at[v],
                    rows_v.at[b, pl.ds(k * _LANES, _LANES)],
                    sem_g.at[b],
                ).wait()

        def write(c, b):
            pltpu.async_copy(
                rows_v.at[b], o_hbm.at[win0 + c], sem_w.at[b]
            )

        def wait_write(c, b):
            pltpu.make_async_copy(
                rows_v.at[b], o_hbm.at[win0 + c], sem_w.at[b]
            ).wait()

        for c in range(_LEAD):
            gather(c, c % _NBUF)

        @pl.loop(0, n_chunk, step=_NBUF)
        def _(jj):
            for bb in range(_NBUF):
                c = jj + bb
                b = bb
                bn = (b + _LEAD) % _NBUF

                @pl.when(c >= _NBUF - _LEAD)
                def _():
                    wait_write(c + _LEAD - _NBUF, bn)

                @pl.when(c + _LEAD < n_chunk)
                def _():
                    gather(c + _LEAD, bn)

                wait_gather(c, b)

                buf = rows_v.at[b]

                @pl.loop(0, _W, step=8)
                def _(r):
                    for dr in range(8):
                        for cc in range(0, _DIM, _LANES):
                            slc = (pl.ds(r + dr, 1), pl.ds(cc, _LANES))
                            buf.at[*slc][...] = buf.at[*slc][...] * _SCALE

                write(c, b)

        for c in range(n_chunk - (_NBUF - _LEAD), n_chunk):
            wait_write(c, c % _NBUF)

    out = emb(lut, idx)
    return out.reshape(*batch_shape, _DIM)
